# Initial kernel scaffold; baseline (speedup 1.0000x reference)
import jax
import jax.numpy as jnp
from jax.experimental import pallas as pl


def kernel(x, edge_index, W1, b1, W2, b2):
    raise NotImplementedError("write your pallas kernel here")


# speedup vs baseline: 23.3679x; 23.3679x over previous
import functools

import jax
import jax.numpy as jnp
from jax import lax
from jax.experimental import pallas as pl
from jax.experimental.pallas import tpu as pltpu
from jax.experimental.pallas import tpu_sc as plsc

_N = 10000
_E = 320000
_F = 128
_H = 128
_CLS = 16

_NC = 2
_NS = 16
_NW = _NC * _NS

_CHUNK = 80
_NCHUNK = _E // (_NW * _CHUNK)

_NPAD = 10240
_STRIPE = _NPAD // _NS

_DEG_PAD = _NPAD
_DEG_STRIPE = _STRIPE

_vmesh = plsc.VectorSubcoreMesh(core_axis_name="c", subcore_axis_name="s")


def _deg_body(dst_hbm, zeros_hbm, out_hbm, dstv, ones_v, deg_sh):
    cid = lax.axis_index("c")
    sid = lax.axis_index("s")
    wid = cid * _NS + sid
    stripe = pl.ds(sid * _DEG_STRIPE, _DEG_STRIPE)
    pltpu.sync_copy(zeros_hbm.at[stripe], deg_sh.at[stripe])
    @pl.loop(0, _CHUNK // 16)
    def _(i):
        ones_v.at[pl.ds(i * 16, 16)][...] = jnp.ones((16,), jnp.float32)
    pltpu.sync_copy(dst_hbm.at[wid], dstv)
    plsc.subcore_barrier()
    @pl.loop(0, _NCHUNK)
    def _(c):
        pltpu.sync_copy(ones_v, deg_sh.at[dstv.at[c]], add=True)
    plsc.subcore_barrier()
    pltpu.sync_copy(deg_sh.at[stripe], out_hbm.at[cid].at[stripe])


def _make_agg(d):
    def body(h_hbm, src_hbm, dst_hbm, zeros_hbm, out_hbm,
             srcv, dstv, rows, s_sh, sem):
        cid = lax.axis_index("c")
        sid = lax.axis_index("s")
        wid = cid * _NS + sid
        stripe = pl.ds(sid * _STRIPE, _STRIPE)
        pltpu.sync_copy(zeros_hbm.at[stripe], s_sh.at[stripe])
        pltpu.sync_copy(src_hbm.at[wid], srcv)
        pltpu.sync_copy(dst_hbm.at[wid], dstv)
        plsc.subcore_barrier()
        @pl.loop(0, _NCHUNK)
        def _(c):
            pltpu.async_copy(h_hbm.at[srcv.at[c]], rows, sem).wait()
            pltpu.sync_copy(rows, s_sh.at[dstv.at[c]], add=True)
        plsc.subcore_barrier()
        pltpu.sync_copy(s_sh.at[stripe], out_hbm.at[cid].at[stripe])

    return pl.kernel(
        body,
        out_type=jax.ShapeDtypeStruct((_NC, _NPAD, d), jnp.float32),
        mesh=_vmesh,
        scratch_types=[
            pltpu.VMEM((_NCHUNK, _CHUNK), jnp.int32),
            pltpu.VMEM((_NCHUNK, _CHUNK), jnp.int32),
            pltpu.VMEM((_CHUNK, d), jnp.float32),
            pltpu.VMEM_SHARED((_NPAD, d), jnp.float32),
            pltpu.SemaphoreType.DMA,
        ],
        compiler_params=pltpu.CompilerParams(use_tc_tiling_on_sc=False),
    )


_deg_call = pl.kernel(
    _deg_body,
    out_type=jax.ShapeDtypeStruct((_NC, _DEG_PAD), jnp.float32),
    mesh=_vmesh,
    scratch_types=[
        pltpu.VMEM((_NCHUNK, _CHUNK), jnp.int32),
        pltpu.VMEM((_CHUNK,), jnp.float32),
        pltpu.VMEM_SHARED((_DEG_PAD,), jnp.float32),
    ],
)

_agg_call_h = _make_agg(_H)
_agg_call_c = _make_agg(_CLS)


def _dinv_from(degT_ref):
    deg = degT_ref[:, 0:1] + degT_ref[:, 1:2] + 1.0
    return lax.rsqrt(deg)


def _tc_b_body(x_ref, w1_ref, degT_ref, h_ref):
    dinv = _dinv_from(degT_ref)
    h = jnp.dot(x_ref[...], w1_ref[...], preferred_element_type=jnp.float32)
    h_ref[...] = h * dinv


def _tc_d_body(s_ref, h1_ref, degT_ref, b1_ref, w2_ref, out_ref):
    dinv = _dinv_from(degT_ref)
    s = s_ref[0, :_N] + s_ref[1, :_N] + h1_ref[...]
    z = jnp.maximum(dinv * s + b1_ref[...], 0.0)
    h2 = jnp.dot(z, w2_ref[...], preferred_element_type=jnp.float32)
    out_ref[...] = h2 * dinv


def _tc_f_body(s2_ref, h2_ref, degT_ref, b2_ref, out_ref):
    dinv = _dinv_from(degT_ref)
    o = dinv * (s2_ref[0, :_N] + s2_ref[1, :_N] + h2_ref[...]) + b2_ref[...]
    m = jnp.max(o, axis=1, keepdims=True)
    lse = jnp.log(jnp.sum(jnp.exp(o - m), axis=1, keepdims=True)) + m
    out_ref[...] = o - lse


_tc_b = pl.pallas_call(
    _tc_b_body, out_shape=jax.ShapeDtypeStruct((_N, _H), jnp.float32))
_tc_d = pl.pallas_call(
    _tc_d_body, out_shape=jax.ShapeDtypeStruct((_N, _CLS), jnp.float32))
_tc_f = pl.pallas_call(
    _tc_f_body, out_shape=jax.ShapeDtypeStruct((_N, _CLS), jnp.float32))


@jax.jit
def _run(x, edge_index, W1, b1, W2, b2):
    ei = edge_index.astype(jnp.int32)
    src = ei[0].reshape(_NW, _NCHUNK, _CHUNK)
    dst = ei[1].reshape(_NW, _NCHUNK, _CHUNK)
    z_deg = jnp.zeros((_DEG_PAD,), jnp.float32)
    z_h = jnp.zeros((_NPAD, _H), jnp.float32)
    z_c = jnp.zeros((_NPAD, _CLS), jnp.float32)

    deg_parts = _deg_call(dst, z_deg)
    degT = deg_parts[:, :_N].T

    h1t = _tc_b(x, W1, degT)
    s1 = _agg_call_h(h1t, src, dst, z_h)
    h2t = _tc_d(s1, h1t, degT, b1.reshape(1, _H), W2)
    s2 = _agg_call_c(h2t, src, dst, z_c)
    return _tc_f(s2, h2t, degT, b2.reshape(1, _CLS))


def kernel(x, edge_index, W1, b1, W2, b2):
    return _run(x, edge_index, W1, b1, W2, b2)
